# baseline (speedup 1.0000x reference)
import functools

import jax
import jax.numpy as jnp
from jax.experimental import pallas as pl
from jax.experimental.pallas import tpu as pltpu

_B, _S, _D, _R, _E = 4, 2048, 2048, 64, 8
_TOPK = 2
_SCALING = 16.0 / _R
_S_TILE = 512
_NEG = -1e30


def _body(eof_ref, xrow_ref, x_ref, a_ref, route_ref, bs_ref, o_ref, cb_ref):
    s = pl.program_id(1)

    @pl.when(s == 0)
    def _routing():
        row = xrow_ref[0, 0]
        logits = [jnp.sum(row * route_ref[e]) for e in range(_E)]
        m1 = logits[0]
        i1 = jnp.int32(0)
        for e in range(1, _E):
            gt = logits[e] > m1
            i1 = jnp.where(gt, jnp.int32(e), i1)
            m1 = jnp.where(gt, logits[e], m1)
        m2 = jnp.float32(_NEG)
        i2 = jnp.int32(0)
        for e in range(_E):
            gt = (logits[e] > m2) & (i1 != e)
            i2 = jnp.where(gt, jnp.int32(e), i2)
            m2 = jnp.where(gt, logits[e], m2)
        e2 = jnp.exp(m2 - m1)
        denom = 1.0 + e2
        g1 = (_SCALING / denom)
        g2 = (_SCALING * e2 / denom)
        cb = jnp.zeros((_D, _R), dtype=jnp.float32)
        for e in range(_E):
            w_e = jnp.where(i1 == e, g1, 0.0) + jnp.where(i2 == e, g2, 0.0)
            cb = cb + w_e * bs_ref[e]
        cb_ref[...] = cb.astype(jnp.bfloat16)

    xt = x_ref[0].astype(jnp.bfloat16)
    shared = jax.lax.dot_general(
        xt, a_ref[...].astype(jnp.bfloat16), (((1,), (1,)), ((), ())),
        preferred_element_type=jnp.float32)
    out = jax.lax.dot_general(
        shared.astype(jnp.bfloat16), cb_ref[...], (((1,), (1,)), ((), ())),
        preferred_element_type=jnp.float32)
    o_ref[0] = out


@jax.jit
def kernel(x, eof_index, lora_A, lora_route, lora_Bs):
    grid = (_B, _S // _S_TILE)
    grid_spec = pltpu.PrefetchScalarGridSpec(
        num_scalar_prefetch=1,
        grid=grid,
        in_specs=[
            pl.BlockSpec((1, 1, _D // 128, 128),
                         lambda b, s, eof: (b, eof[b], 0, 0)),
            pl.BlockSpec((1, _S_TILE, _D), lambda b, s, eof: (b, s, 0)),
            pl.BlockSpec((_R, _D), lambda b, s, eof: (0, 0)),
            pl.BlockSpec((_E, _D // 128, 128), lambda b, s, eof: (0, 0, 0)),
            pl.BlockSpec((_E, _D, _R), lambda b, s, eof: (0, 0, 0)),
        ],
        out_specs=pl.BlockSpec((1, _S_TILE, _D), lambda b, s, eof: (b, s, 0)),
        scratch_shapes=[pltpu.VMEM((_D, _R), jnp.bfloat16)],
    )
    return pl.pallas_call(
        _body,
        grid_spec=grid_spec,
        out_shape=jax.ShapeDtypeStruct((_B, _S, _D), jnp.float32),
    )(eof_index, x.reshape(_B, _S, _D // 128, 128), x,
      lora_A, lora_route.reshape(_E, _D // 128, 128), lora_Bs)


# speedup vs baseline: 1.8830x; 1.8830x over previous
import functools

import jax
import jax.numpy as jnp
from jax.experimental import pallas as pl
from jax.experimental.pallas import tpu as pltpu

_B, _S, _D, _R, _E = 4, 2048, 2048, 64, 8
_TOPK = 2
_SCALING = 16.0 / _R
_S_TILE = 512
_NEG = -1e30


def _body(eof_ref, xrow_ref, x_ref, a_ref, route_ref, bs_ref, o_ref, cb_ref):
    b = pl.program_id(0)
    s = pl.program_id(1)

    @pl.when(s == 0)
    def _routing():
        sub = eof_ref[b] % 8
        blk = xrow_ref[0]
        iota = jax.lax.broadcasted_iota(jnp.int32, (8, _D), 0)
        row = jnp.sum(jnp.where(iota == sub, blk, 0.0), axis=0,
                      keepdims=True)
        logits2d = jax.lax.dot_general(
            row, route_ref[...], (((1,), (1,)), ((), ())),
            preferred_element_type=jnp.float32)
        logits = [logits2d[0, e] for e in range(_E)]
        m1 = logits[0]
        i1 = jnp.int32(0)
        for e in range(1, _E):
            gt = logits[e] > m1
            i1 = jnp.where(gt, jnp.int32(e), i1)
            m1 = jnp.where(gt, logits[e], m1)
        m2 = jnp.float32(_NEG)
        i2 = jnp.int32(0)
        for e in range(_E):
            gt = (logits[e] > m2) & (i1 != e)
            i2 = jnp.where(gt, jnp.int32(e), i2)
            m2 = jnp.where(gt, logits[e], m2)
        e2 = jnp.exp(m2 - m1)
        denom = 1.0 + e2
        g1 = (_SCALING / denom)
        g2 = (_SCALING * e2 / denom)
        cb = jnp.zeros((_D, _R), dtype=jnp.float32)
        for e in range(_E):
            w_e = jnp.where(i1 == e, g1, 0.0) + jnp.where(i2 == e, g2, 0.0)
            cb = cb + w_e * bs_ref[e]
        cb_ref[...] = cb.astype(jnp.bfloat16)

    xt = x_ref[0].astype(jnp.bfloat16)
    shared = jax.lax.dot_general(
        xt, a_ref[...].astype(jnp.bfloat16), (((1,), (1,)), ((), ())),
        preferred_element_type=jnp.float32)
    out = jax.lax.dot_general(
        shared.astype(jnp.bfloat16), cb_ref[...], (((1,), (1,)), ((), ())),
        preferred_element_type=jnp.float32)
    o_ref[0] = out


@jax.jit
def kernel(x, eof_index, lora_A, lora_route, lora_Bs):
    grid = (_B, _S // _S_TILE)
    grid_spec = pltpu.PrefetchScalarGridSpec(
        num_scalar_prefetch=1,
        grid=grid,
        in_specs=[
            pl.BlockSpec((1, 8, _D), lambda b, s, eof: (b, eof[b] // 8, 0)),
            pl.BlockSpec((1, _S_TILE, _D), lambda b, s, eof: (b, s, 0)),
            pl.BlockSpec((_R, _D), lambda b, s, eof: (0, 0)),
            pl.BlockSpec((_E, _D), lambda b, s, eof: (0, 0)),
            pl.BlockSpec((_E, _D, _R), lambda b, s, eof: (0, 0, 0)),
        ],
        out_specs=pl.BlockSpec((1, _S_TILE, _D), lambda b, s, eof: (b, s, 0)),
        scratch_shapes=[pltpu.VMEM((_D, _R), jnp.bfloat16)],
    )
    return pl.pallas_call(
        _body,
        grid_spec=grid_spec,
        out_shape=jax.ShapeDtypeStruct((_B, _S, _D), jnp.float32),
    )(eof_index, x, x, lora_A, lora_route, lora_Bs)


# speedup vs baseline: 1.9775x; 1.0502x over previous
import functools

import jax
import jax.numpy as jnp
from jax.experimental import pallas as pl
from jax.experimental.pallas import tpu as pltpu

_B, _S, _D, _R, _E = 4, 2048, 2048, 64, 8
_TOPK = 2
_SCALING = 16.0 / _R
_S_TILE = 1024
_NEG = -1e30


def _body(eof_ref, xrow_ref, x_ref, a_ref, route_ref, bs_ref, o_ref, cb_ref):
    b = pl.program_id(0)
    s = pl.program_id(1)

    @pl.when(s == 0)
    def _routing():
        sub = eof_ref[b] % 8
        blk = xrow_ref[0]
        iota = jax.lax.broadcasted_iota(jnp.int32, (8, _D), 0)
        row = jnp.sum(jnp.where(iota == sub, blk, 0.0), axis=0,
                      keepdims=True)
        logits2d = jax.lax.dot_general(
            row, route_ref[...], (((1,), (1,)), ((), ())),
            preferred_element_type=jnp.float32)
        logits = [logits2d[0, e] for e in range(_E)]
        m1 = logits[0]
        i1 = jnp.int32(0)
        for e in range(1, _E):
            gt = logits[e] > m1
            i1 = jnp.where(gt, jnp.int32(e), i1)
            m1 = jnp.where(gt, logits[e], m1)
        m2 = jnp.float32(_NEG)
        i2 = jnp.int32(0)
        for e in range(_E):
            gt = (logits[e] > m2) & (i1 != e)
            i2 = jnp.where(gt, jnp.int32(e), i2)
            m2 = jnp.where(gt, logits[e], m2)
        e2 = jnp.exp(m2 - m1)
        denom = 1.0 + e2
        g1 = (_SCALING / denom)
        g2 = (_SCALING * e2 / denom)
        cb = jnp.zeros((_D, _R), dtype=jnp.float32)
        for e in range(_E):
            w_e = jnp.where(i1 == e, g1, 0.0) + jnp.where(i2 == e, g2, 0.0)
            cb = cb + w_e * bs_ref[e]
        cb_ref[...] = cb.astype(jnp.bfloat16)

    xt = x_ref[0].astype(jnp.bfloat16)
    shared = jax.lax.dot_general(
        xt, a_ref[...].astype(jnp.bfloat16), (((1,), (1,)), ((), ())),
        preferred_element_type=jnp.float32)
    out = jax.lax.dot_general(
        shared.astype(jnp.bfloat16), cb_ref[...], (((1,), (1,)), ((), ())),
        preferred_element_type=jnp.float32)
    o_ref[0] = out


@jax.jit
def kernel(x, eof_index, lora_A, lora_route, lora_Bs):
    grid = (_B, _S // _S_TILE)
    grid_spec = pltpu.PrefetchScalarGridSpec(
        num_scalar_prefetch=1,
        grid=grid,
        in_specs=[
            pl.BlockSpec((1, 8, _D), lambda b, s, eof: (b, eof[b] // 8, 0)),
            pl.BlockSpec((1, _S_TILE, _D), lambda b, s, eof: (b, s, 0)),
            pl.BlockSpec((_R, _D), lambda b, s, eof: (0, 0)),
            pl.BlockSpec((_E, _D), lambda b, s, eof: (0, 0)),
            pl.BlockSpec((_E, _D, _R), lambda b, s, eof: (0, 0, 0)),
        ],
        out_specs=pl.BlockSpec((1, _S_TILE, _D), lambda b, s, eof: (b, s, 0)),
        scratch_shapes=[pltpu.VMEM((_D, _R), jnp.bfloat16)],
    )
    return pl.pallas_call(
        _body,
        grid_spec=grid_spec,
        out_shape=jax.ShapeDtypeStruct((_B, _S, _D), jnp.float32),
    )(eof_index, x, x, lora_A, lora_route, lora_Bs)
